# baseline (speedup 1.0000x reference)
import functools

import jax
import jax.numpy as jnp
from jax.experimental import pallas as pl
from jax.experimental.pallas import tpu as pltpu

TOPK_FRAC = 0.3
BLK_C = 2048


def _ohem_kernel(x_ref, t_ref, o_ref, m_ref, s_ref, g_ref, *, c_total, n_blk, k):
    j = pl.program_id(0)

    @pl.when(j == 0)
    def _init():
        m_ref[...] = jnp.full_like(m_ref, -jnp.inf)
        s_ref[...] = jnp.zeros_like(s_ref)
        g_ref[...] = jnp.full_like(g_ref, -jnp.inf)

    x = x_ref[...]
    b, blk_c = x.shape
    col = jax.lax.broadcasted_iota(jnp.int32, (b, blk_c), 1) + j * blk_c
    valid = col < c_total
    xm = jnp.where(valid, x, -jnp.inf)

    m_old = m_ref[...]
    m_new = jnp.maximum(m_old, jnp.max(xm, axis=1, keepdims=True))
    e = jnp.where(valid, jnp.exp(x - m_new), 0.0)
    s_ref[...] = s_ref[...] * jnp.exp(m_old - m_new) + jnp.sum(
        e, axis=1, keepdims=True
    )
    m_ref[...] = m_new

    tgt = t_ref[...]
    g_ref[...] = jnp.maximum(
        g_ref[...], jnp.max(jnp.where(col == tgt, x, -jnp.inf), axis=1, keepdims=True)
    )

    @pl.when(j == n_blk - 1)
    def _finish():
        loss = m_ref[...] + jnp.log(s_ref[...]) - g_ref[...]
        u = jax.lax.bitcast_convert_type(loss, jnp.uint32)
        sortable = u ^ jnp.where(
            (u >> 31) > 0, jnp.uint32(0xFFFFFFFF), jnp.uint32(0x80000000)
        )

        def body(i, th):
            cand = th | (jnp.uint32(1) << (31 - i))
            cnt = jnp.sum((sortable >= cand).astype(jnp.int32))
            return jnp.where(cnt >= k, cand, th)

        th = jax.lax.fori_loop(0, 32, body, jnp.uint32(0), unroll=True)
        gt = sortable > th
        cnt_gt = jnp.sum(gt.astype(jnp.int32))
        sum_gt = jnp.sum(jnp.where(gt, loss, 0.0))
        kth_val = jnp.max(jnp.where(sortable == th, loss, -jnp.inf))
        total = sum_gt + (k - cnt_gt).astype(jnp.float32) * kth_val
        o_ref[...] = jnp.full_like(o_ref, total / k)


def kernel(inputs, targets):
    b, c = inputs.shape
    k = max(1, int(b * TOPK_FRAC))
    n_blk = pl.cdiv(c, BLK_C)
    tgt2d = targets.reshape(b, 1)

    out = pl.pallas_call(
        functools.partial(_ohem_kernel, c_total=c, n_blk=n_blk, k=k),
        grid=(n_blk,),
        in_specs=[
            pl.BlockSpec((b, BLK_C), lambda j: (0, j)),
            pl.BlockSpec((b, 1), lambda j: (0, 0)),
        ],
        out_specs=pl.BlockSpec((1, 1), lambda j: (0, 0)),
        out_shape=jax.ShapeDtypeStruct((1, 1), jnp.float32),
        scratch_shapes=[
            pltpu.VMEM((b, 1), jnp.float32),
            pltpu.VMEM((b, 1), jnp.float32),
            pltpu.VMEM((b, 1), jnp.float32),
        ],
    )(inputs, tgt2d)
    return out.reshape(())
